# baseline (speedup 1.0000x reference)
import functools

import jax
import jax.numpy as jnp
from jax import lax
from jax.experimental import pallas as pl
from jax.experimental.pallas import tpu as pltpu
from jax.experimental.pallas import tpu_sc as plsc

NC = 2
NS = 16
L = 16
CW = 16


def _sc_count_dst(dst, n):
    e = dst.shape[0]
    nw = NC * NS
    ew = e // nw
    k = 80
    ch = ew // k
    assert ew * nw == e and ch * k == ew and ch >= 4
    wt = 10
    rt = n // wt
    assert rt * wt == n and rt % 8 == 0
    mesh = plsc.VectorSubcoreMesh(core_axis_name="c", subcore_axis_name="s")

    @functools.partial(
        pl.kernel,
        mesh=mesh,
        out_type=jax.ShapeDtypeStruct((NC, n, CW), jnp.float32),
        compiler_params=pltpu.CompilerParams(use_tc_tiling_on_sc=False),
        scratch_types=[
            pltpu.VMEM((k,), jnp.int32),
            pltpu.VMEM((k,), jnp.int32),
            pltpu.VMEM((k,), jnp.int32),
            pltpu.VMEM((k, CW), jnp.float32),
            pltpu.VMEM((rt, CW), jnp.float32),
            pltpu.VMEM_SHARED((n, CW), jnp.float32),
            pltpu.SemaphoreType.DMA,
            pltpu.SemaphoreType.DMA,
            pltpu.SemaphoreType.DMA,
            pltpu.SemaphoreType.DMA,
            pltpu.SemaphoreType.DMA,
            pltpu.SemaphoreType.DMA,
        ],
    )
    def body(dst_hbm, out_hbm, dstv0, dstv1, dstv2, ones, zbuf, acc,
             semd0, semd1, semd2, sems0, sems1, sems2):
        c = lax.axis_index("c")
        s = lax.axis_index("s")
        w = s * NC + c
        dstv = (dstv0, dstv1, dstv2)
        semd = (semd0, semd1, semd2)
        sems = (sems0, sems1, sems2)

        @pl.loop(0, k)
        def _(i):
            ones[i] = jnp.ones((CW,), jnp.float32)

        @pl.loop(0, rt)
        def _(i):
            zbuf[i] = jnp.zeros((CW,), jnp.float32)

        @pl.when(s < wt)
        def _():
            pltpu.sync_copy(zbuf, acc.at[pl.ds(s * rt, rt)])

        def issue(j, b):
            pltpu.async_copy(dst_hbm.at[pl.ds(w * ew + j * k, k)],
                             dstv[b], semd[b])

        issue(0, 0)
        issue(1, 1)
        plsc.subcore_barrier()

        @pl.loop(0, (ch + 2) // 3)
        def _(gi):
            for u in range(3):
                i = gi * 3 + u
                b2 = (u + 2) % 3

                @pl.when(i < ch)
                def _():
                    pltpu.make_async_copy(dst_hbm.at[pl.ds(0, k)],
                                          dstv[u], semd[u]).wait()
                    pltpu.async_copy(ones, acc.at[dstv[u]], sems[u], add=True)

                    @pl.when(i + 2 < ch)
                    def _():
                        @pl.when(i >= 1)
                        def _():
                            pltpu.make_async_copy(
                                ones, acc.at[dstv[b2]], sems[b2]).wait()

                        issue(i + 2, b2)

        for b in ((ch - 3) % 3, (ch - 2) % 3, (ch - 1) % 3):
            pltpu.make_async_copy(ones, acc.at[dstv[b]], sems[b]).wait()
        plsc.subcore_barrier()

        @pl.when(s < wt)
        def _():
            pltpu.sync_copy(acc.at[pl.ds(s * rt, rt)],
                            out_hbm.at[c, pl.ds(s * rt, rt)])

    return body(dst)


def _sc_scatter_rows(g, src, dst, zeros):
    n, d = g.shape
    e = src.shape[0]
    nw = NC * NS
    k = 128
    tch = e // k
    assert tch * k == e and tch >= 3 * nw
    wt = 10
    rt = n // wt
    assert rt * wt == n and rt % 8 == 0 and zeros.shape == (rt, d)
    mesh = plsc.VectorSubcoreMesh(core_axis_name="c", subcore_axis_name="s")

    @functools.partial(
        pl.kernel,
        mesh=mesh,
        out_type=jax.ShapeDtypeStruct((NC, n, d), jnp.float32),
        scratch_types=[
            pltpu.VMEM((k,), jnp.int32),
            pltpu.VMEM((k,), jnp.int32),
            pltpu.VMEM((k,), jnp.int32),
            pltpu.VMEM((k,), jnp.int32),
            pltpu.VMEM((k,), jnp.int32),
            pltpu.VMEM((k,), jnp.int32),
            pltpu.VMEM((k,), jnp.int32),
            pltpu.VMEM((k,), jnp.int32),
            pltpu.VMEM((k,), jnp.int32),
            pltpu.VMEM((k, d), jnp.float32),
            pltpu.VMEM((k, d), jnp.float32),
            pltpu.VMEM((k, d), jnp.float32),
            pltpu.VMEM_SHARED((n, d), jnp.float32),
            pltpu.SemaphoreType.DMA,
            pltpu.SemaphoreType.DMA,
            pltpu.SemaphoreType.DMA,
            pltpu.SemaphoreType.DMA,
            pltpu.SemaphoreType.DMA,
            pltpu.SemaphoreType.DMA,
            pltpu.SemaphoreType.DMA,
            pltpu.SemaphoreType.DMA,
            pltpu.SemaphoreType.DMA,
        ],
    )
    def body(g_hbm, src_hbm, dst_hbm, zero_hbm, out_hbm,
             srcv0, srcv1, srcv2, dstv0, dstv1, dstv2, dsts0, dsts1, dsts2,
             rows0, rows1, rows2, acc, semi0, semi1, semi2,
             semg0, semg1, semg2, sems0, sems1, sems2):
        c = lax.axis_index("c")
        s = lax.axis_index("s")
        w = s * NC + c
        cs = (w * tch) // nw
        cn = ((w + 1) * tch) // nw - cs
        srcv = (srcv0, srcv1, srcv2)
        dstv = (dstv0, dstv1, dstv2)
        dsts = (dsts0, dsts1, dsts2)
        rows = (rows0, rows1, rows2)
        semi = (semi0, semi1, semi2)
        semg = (semg0, semg1, semg2)
        sems = (sems0, sems1, sems2)

        @pl.when(s < wt)
        def _():
            pltpu.sync_copy(zero_hbm, acc.at[pl.ds(s * rt, rt)])

        def issue_idx(j, b):
            pltpu.async_copy(src_hbm.at[pl.ds((cs + j) * k, k)],
                             srcv[b], semi[b])
            pltpu.async_copy(dst_hbm.at[pl.ds((cs + j) * k, k)],
                             dstv[b], semi[b])

        def wait_idx(b):
            pltpu.make_async_copy(src_hbm.at[pl.ds(0, k)],
                                  srcv[b], semi[b]).wait()
            pltpu.make_async_copy(dst_hbm.at[pl.ds(0, k)],
                                  dstv[b], semi[b]).wait()

        def start_gather(b):
            wait_idx(b)
            for q in range(k // L):
                dsts[b][pl.ds(q * L, L)] = dstv[b][pl.ds(q * L, L)]
            pltpu.async_copy(g_hbm.at[srcv[b]], rows[b], semg[b])

        issue_idx(0, 0)
        issue_idx(1, 1)
        start_gather(0)
        plsc.subcore_barrier()

        @pl.loop(0, (cn + 2) // 3)
        def _(gi):
            for u in range(3):
                i = gi * 3 + u
                u1 = (u + 1) % 3
                u2 = (u + 2) % 3

                @pl.when(i < cn)
                def _():
                    pltpu.make_async_copy(g_hbm.at[srcv[u]],
                                          rows[u], semg[u]).wait()
                    pltpu.async_copy(rows[u], acc.at[dsts[u]], sems[u],
                                     add=True)

                    @pl.when(i + 1 < cn)
                    def _():
                        @pl.when(i >= 2)
                        def _():
                            pltpu.make_async_copy(
                                rows[u1], acc.at[dsts[u1]], sems[u1]).wait()

                        start_gather(u1)

                    @pl.when(i + 2 < cn)
                    def _():
                        issue_idx(i + 2, u2)

        for b in range(3):
            pltpu.make_async_copy(rows[b], acc.at[dsts[b]], sems[b]).wait()
        plsc.subcore_barrier()

        @pl.when(s < wt)
        def _():
            pltpu.sync_copy(acc.at[pl.ds(s * rt, rt)],
                            out_hbm.at[c, pl.ds(s * rt, rt)])

    return body(g, src, dst, zeros)


def _dis_block(c0, c1):
    deg = c0[:, :1] + c1[:, :1] + 1.0
    return lax.rsqrt(deg)


def _tc_pre(x, w1, c0, c1):
    n, d = x.shape
    rb = 2000
    grid = n // rb

    def body(x_r, w_r, c0_r, c1_r, g1_r):
        dis = _dis_block(c0_r[...], c1_r[...])
        g1_r[...] = dis * jnp.dot(x_r[...], w_r[...],
                                  preferred_element_type=jnp.float32)

    return pl.pallas_call(
        body,
        grid=(grid,),
        in_specs=[
            pl.BlockSpec((rb, d), lambda i: (i, 0)),
            pl.BlockSpec((d, d), lambda i: (0, 0)),
            pl.BlockSpec((rb, CW), lambda i: (i, 0)),
            pl.BlockSpec((rb, CW), lambda i: (i, 0)),
        ],
        out_specs=pl.BlockSpec((rb, d), lambda i: (i, 0)),
        out_shape=jax.ShapeDtypeStruct((n, d), jnp.float32),
    )(x, w1, c0, c1)


def _tc_mid(s0, s1, g1, c0, c1, b1, xroot, w2):
    n, d = g1.shape
    rb = 2000
    grid = n // rb

    def body(s0_r, s1_r, g1_r, c0_r, c1_r, b1_r, xr_r, w2_r, conv1_r, g2_r):
        dis = _dis_block(c0_r[...], c1_r[...])
        conv1 = dis * (s0_r[...] + s1_r[...] + g1_r[...]) + b1_r[...]
        conv1_r[...] = conv1
        w2 = w2_r[...]
        h2 = (jnp.dot(jnp.maximum(conv1, 0.0), w2[d:],
                      preferred_element_type=jnp.float32)
              + jnp.dot(jnp.maximum(xr_r[...], 0.0), w2[:d],
                        preferred_element_type=jnp.float32))
        g2_r[...] = dis * h2

    return pl.pallas_call(
        body,
        grid=(grid,),
        in_specs=[
            pl.BlockSpec((rb, d), lambda i: (i, 0)),
            pl.BlockSpec((rb, d), lambda i: (i, 0)),
            pl.BlockSpec((rb, d), lambda i: (i, 0)),
            pl.BlockSpec((rb, CW), lambda i: (i, 0)),
            pl.BlockSpec((rb, CW), lambda i: (i, 0)),
            pl.BlockSpec((1, d), lambda i: (0, 0)),
            pl.BlockSpec((1, d), lambda i: (0, 0)),
            pl.BlockSpec((2 * d, d), lambda i: (0, 0)),
        ],
        out_specs=[
            pl.BlockSpec((rb, d), lambda i: (i, 0)),
            pl.BlockSpec((rb, d), lambda i: (i, 0)),
        ],
        out_shape=[
            jax.ShapeDtypeStruct((n, d), jnp.float32),
            jax.ShapeDtypeStruct((n, d), jnp.float32),
        ],
    )(s0, s1, g1, c0, c1, b1, xroot, w2)


def _tc_final(s0, s1, g2, c0, c1, b2):
    n, d = g2.shape
    rb = 2000
    grid = n // rb

    def body(s0_r, s1_r, g2_r, c0_r, c1_r, b2_r, out_r):
        i = pl.program_id(0)
        dis = _dis_block(c0_r[...], c1_r[...])
        conv2 = jnp.maximum(dis * (s0_r[...] + s1_r[...] + g2_r[...]) + b2_r[...], 0.0)
        part = jnp.sum(conv2, axis=0, keepdims=True) * (1.0 / n)

        @pl.when(i == 0)
        def _():
            out_r[...] = part

        @pl.when(i > 0)
        def _():
            out_r[...] = out_r[...] + part

    return pl.pallas_call(
        body,
        grid=(grid,),
        in_specs=[
            pl.BlockSpec((rb, d), lambda i: (i, 0)),
            pl.BlockSpec((rb, d), lambda i: (i, 0)),
            pl.BlockSpec((rb, d), lambda i: (i, 0)),
            pl.BlockSpec((rb, CW), lambda i: (i, 0)),
            pl.BlockSpec((rb, CW), lambda i: (i, 0)),
            pl.BlockSpec((1, d), lambda i: (0, 0)),
        ],
        out_specs=pl.BlockSpec((1, d), lambda i: (0, 0)),
        out_shape=jax.ShapeDtypeStruct((1, d), jnp.float32),
    )(s0, s1, g2, c0, c1, b2)


def kernel(x, edge_index, rootIndex, W1, b1, W2, b2):
    n, d = x.shape
    src = edge_index[0]
    dst = edge_index[1]

    counts = _sc_count_dst(dst, n)
    c0, c1 = counts[0], counts[1]
    g1 = _tc_pre(x, W1, c0, c1)
    zeros = jnp.zeros((n // 10, d), jnp.float32)
    parts1 = _sc_scatter_rows(g1, src, dst, zeros)
    xroot = lax.dynamic_slice_in_dim(x, rootIndex, 1, axis=0)
    conv1, g2 = _tc_mid(parts1[0], parts1[1], g1, c0, c1,
                        b1.reshape(1, -1), xroot, W2)
    parts2 = _sc_scatter_rows(g2, src, dst, zeros)
    mean2 = _tc_final(parts2[0], parts2[1], g2, c0, c1, b2.reshape(1, -1))
    root1 = lax.dynamic_slice_in_dim(conv1, rootIndex, 1, axis=0)
    return jnp.concatenate([root1, mean2], axis=1)


# speedup vs baseline: 1.2088x; 1.2088x over previous
import functools

import jax
import jax.numpy as jnp
from jax import lax
from jax.experimental import pallas as pl
from jax.experimental.pallas import tpu as pltpu
from jax.experimental.pallas import tpu_sc as plsc

NC = 2
NS = 16
L = 16
CW = 16


def _sc_count_dst(dst, n):
    e = dst.shape[0]
    nw = NC * NS
    k = 128
    tch = e // k
    assert tch * k == e and tch >= 3 * nw
    wt = 10
    rt = n // wt
    assert rt * wt == n and rt % 8 == 0
    mesh = plsc.VectorSubcoreMesh(core_axis_name="c", subcore_axis_name="s")

    @functools.partial(
        pl.kernel,
        mesh=mesh,
        out_type=jax.ShapeDtypeStruct((NC, n, CW), jnp.float32),
        compiler_params=pltpu.CompilerParams(use_tc_tiling_on_sc=False),
        scratch_types=[
            pltpu.VMEM((k,), jnp.int32),
            pltpu.VMEM((k,), jnp.int32),
            pltpu.VMEM((k,), jnp.int32),
            pltpu.VMEM((k, CW), jnp.float32),
            pltpu.VMEM((rt, CW), jnp.float32),
            pltpu.VMEM_SHARED((n, CW), jnp.float32),
            pltpu.SemaphoreType.DMA,
            pltpu.SemaphoreType.DMA,
            pltpu.SemaphoreType.DMA,
            pltpu.SemaphoreType.DMA,
            pltpu.SemaphoreType.DMA,
            pltpu.SemaphoreType.DMA,
        ],
    )
    def body(dst_hbm, out_hbm, dstv0, dstv1, dstv2, ones, zbuf, acc,
             semd0, semd1, semd2, sems0, sems1, sems2):
        c = lax.axis_index("c")
        s = lax.axis_index("s")
        w = s * NC + c
        cs = (w * tch) // nw
        cn = ((w + 1) * tch) // nw - cs
        dstv = (dstv0, dstv1, dstv2)
        semd = (semd0, semd1, semd2)
        sems = (sems0, sems1, sems2)

        @pl.loop(0, k)
        def _(i):
            ones[i] = jnp.ones((CW,), jnp.float32)

        @pl.loop(0, rt)
        def _(i):
            zbuf[i] = jnp.zeros((CW,), jnp.float32)

        @pl.when(s < wt)
        def _():
            pltpu.sync_copy(zbuf, acc.at[pl.ds(s * rt, rt)])

        def issue(j, b):
            pltpu.async_copy(dst_hbm.at[pl.ds((cs + j) * k, k)],
                             dstv[b], semd[b])

        issue(0, 0)
        issue(1, 1)
        plsc.subcore_barrier()

        @pl.loop(0, (cn + 2) // 3)
        def _(gi):
            for u in range(3):
                i = gi * 3 + u
                b2 = (u + 2) % 3

                @pl.when(i < cn)
                def _():
                    pltpu.make_async_copy(dst_hbm.at[pl.ds(0, k)],
                                          dstv[u], semd[u]).wait()
                    pltpu.async_copy(ones, acc.at[dstv[u]], sems[u], add=True)

                    @pl.when(i + 2 < cn)
                    def _():
                        @pl.when(i >= 1)
                        def _():
                            pltpu.make_async_copy(
                                ones, acc.at[dstv[b2]], sems[b2]).wait()

                        issue(i + 2, b2)

        for b in range(3):
            pltpu.make_async_copy(ones, acc.at[dstv[b]], sems[b]).wait()
        plsc.subcore_barrier()

        @pl.when(s < wt)
        def _():
            pltpu.sync_copy(acc.at[pl.ds(s * rt, rt)],
                            out_hbm.at[c, pl.ds(s * rt, rt)])

    return body(dst)


def _sc_scatter_rows(g, src, dst):
    n, d = g.shape
    e = src.shape[0]
    nw = NC * NS
    ew = e // nw
    k = 80
    ch = ew // k
    assert ew * nw == e and ch * k == ew and ch >= 4
    wt = 10
    rt = n // wt
    zr = 40
    assert rt * wt == n and rt % zr == 0 and zr % 8 == 0
    mesh = plsc.VectorSubcoreMesh(core_axis_name="c", subcore_axis_name="s")

    @functools.partial(
        pl.kernel,
        mesh=mesh,
        out_type=jax.ShapeDtypeStruct((NC, n, d), jnp.float32),
        scratch_types=[
            pltpu.VMEM((ew,), jnp.int32),
            pltpu.VMEM((k,), jnp.int32),
            pltpu.VMEM((k,), jnp.int32),
            pltpu.VMEM((k,), jnp.int32),
            pltpu.VMEM((k, d), jnp.float32),
            pltpu.VMEM((k, d), jnp.float32),
            pltpu.VMEM((k, d), jnp.float32),
            pltpu.VMEM((zr, d), jnp.float32),
            pltpu.VMEM_SHARED((n, d), jnp.float32),
            pltpu.SemaphoreType.DMA,
            pltpu.SemaphoreType.DMA,
            pltpu.SemaphoreType.DMA,
            pltpu.SemaphoreType.DMA,
            pltpu.SemaphoreType.DMA,
            pltpu.SemaphoreType.DMA,
            pltpu.SemaphoreType.DMA,
            pltpu.SemaphoreType.DMA,
            pltpu.SemaphoreType.DMA,
        ],
    )
    def body(g_hbm, src_hbm, dst_hbm, out_hbm, srcall, dstv0, dstv1, dstv2,
             rows0, rows1, rows2, zbuf, acc, semd0, semd1, semd2,
             semg0, semg1, semg2, sems0, sems1, sems2):
        c = lax.axis_index("c")
        s = lax.axis_index("s")
        w = s * NC + c
        cols = d // L
        dstv = (dstv0, dstv1, dstv2)
        rows = (rows0, rows1, rows2)
        semd = (semd0, semd1, semd2)
        semg = (semg0, semg1, semg2)
        sems = (sems0, sems1, sems2)

        @pl.loop(0, zr * cols)
        def _(i):
            zbuf[i // cols, pl.ds((i % cols) * L, L)] = jnp.zeros((L,), jnp.float32)

        @pl.when(s < wt)
        def _():
            for r in range(rt // zr):
                pltpu.sync_copy(zbuf, acc.at[pl.ds(s * rt + r * zr, zr)])

        pltpu.sync_copy(src_hbm.at[pl.ds(w * ew, ew)], srcall)

        def issue(j, b):
            pltpu.async_copy(dst_hbm.at[pl.ds(w * ew + j * k, k)],
                             dstv[b], semd[b])
            pltpu.async_copy(g_hbm.at[srcall.at[pl.ds(j * k, k)]],
                             rows[b], semg[b])

        issue(0, 0)
        issue(1, 1)
        plsc.subcore_barrier()

        @pl.loop(0, (ch + 2) // 3)
        def _(gi):
            for u in range(3):
                i = gi * 3 + u
                b2 = (u + 2) % 3

                @pl.when(i < ch)
                def _():
                    pltpu.make_async_copy(dst_hbm.at[pl.ds(0, k)],
                                          dstv[u], semd[u]).wait()
                    pltpu.make_async_copy(g_hbm.at[srcall.at[pl.ds(0, k)]],
                                          rows[u], semg[u]).wait()
                    pltpu.async_copy(rows[u], acc.at[dstv[u]], sems[u],
                                     add=True)

                    @pl.when(i + 2 < ch)
                    def _():
                        @pl.when(i >= 1)
                        def _():
                            pltpu.make_async_copy(
                                rows[b2], acc.at[dstv[b2]], sems[b2]).wait()

                        issue(i + 2, b2)

        for b in ((ch - 3) % 3, (ch - 2) % 3, (ch - 1) % 3):
            pltpu.make_async_copy(rows[b], acc.at[dstv[b]], sems[b]).wait()
        plsc.subcore_barrier()

        @pl.when(s < wt)
        def _():
            pltpu.sync_copy(acc.at[pl.ds(s * rt, rt)],
                            out_hbm.at[c, pl.ds(s * rt, rt)])

    return body(g, src, dst)


def _dis_block(c0, c1):
    deg = c0[:, :1] + c1[:, :1] + 1.0
    return lax.rsqrt(deg)


def _tc_pre(x, w1, c0, c1):
    n, d = x.shape
    rb = 2000
    grid = n // rb

    def body(x_r, w_r, c0_r, c1_r, g1_r):
        dis = _dis_block(c0_r[...], c1_r[...])
        g1_r[...] = dis * jnp.dot(x_r[...], w_r[...],
                                  preferred_element_type=jnp.float32)

    return pl.pallas_call(
        body,
        grid=(grid,),
        in_specs=[
            pl.BlockSpec((rb, d), lambda i: (i, 0)),
            pl.BlockSpec((d, d), lambda i: (0, 0)),
            pl.BlockSpec((rb, CW), lambda i: (i, 0)),
            pl.BlockSpec((rb, CW), lambda i: (i, 0)),
        ],
        out_specs=pl.BlockSpec((rb, d), lambda i: (i, 0)),
        out_shape=jax.ShapeDtypeStruct((n, d), jnp.float32),
    )(x, w1, c0, c1)


def _tc_mid(s0, s1, g1, c0, c1, b1, xroot, w2):
    n, d = g1.shape
    rb = 2000
    grid = n // rb

    def body(s0_r, s1_r, g1_r, c0_r, c1_r, b1_r, xr_r, w2_r, conv1_r, g2_r):
        dis = _dis_block(c0_r[...], c1_r[...])
        conv1 = dis * (s0_r[...] + s1_r[...] + g1_r[...]) + b1_r[...]
        conv1_r[...] = conv1
        w2 = w2_r[...]
        h2 = (jnp.dot(jnp.maximum(conv1, 0.0), w2[d:],
                      preferred_element_type=jnp.float32)
              + jnp.dot(jnp.maximum(xr_r[...], 0.0), w2[:d],
                        preferred_element_type=jnp.float32))
        g2_r[...] = dis * h2

    return pl.pallas_call(
        body,
        grid=(grid,),
        in_specs=[
            pl.BlockSpec((rb, d), lambda i: (i, 0)),
            pl.BlockSpec((rb, d), lambda i: (i, 0)),
            pl.BlockSpec((rb, d), lambda i: (i, 0)),
            pl.BlockSpec((rb, CW), lambda i: (i, 0)),
            pl.BlockSpec((rb, CW), lambda i: (i, 0)),
            pl.BlockSpec((1, d), lambda i: (0, 0)),
            pl.BlockSpec((1, d), lambda i: (0, 0)),
            pl.BlockSpec((2 * d, d), lambda i: (0, 0)),
        ],
        out_specs=[
            pl.BlockSpec((rb, d), lambda i: (i, 0)),
            pl.BlockSpec((rb, d), lambda i: (i, 0)),
        ],
        out_shape=[
            jax.ShapeDtypeStruct((n, d), jnp.float32),
            jax.ShapeDtypeStruct((n, d), jnp.float32),
        ],
    )(s0, s1, g1, c0, c1, b1, xroot, w2)


def _tc_final(s0, s1, g2, c0, c1, b2):
    n, d = g2.shape
    rb = 2000
    grid = n // rb

    def body(s0_r, s1_r, g2_r, c0_r, c1_r, b2_r, out_r):
        i = pl.program_id(0)
        dis = _dis_block(c0_r[...], c1_r[...])
        conv2 = jnp.maximum(dis * (s0_r[...] + s1_r[...] + g2_r[...]) + b2_r[...], 0.0)
        part = jnp.sum(conv2, axis=0, keepdims=True) * (1.0 / n)

        @pl.when(i == 0)
        def _():
            out_r[...] = part

        @pl.when(i > 0)
        def _():
            out_r[...] = out_r[...] + part

    return pl.pallas_call(
        body,
        grid=(grid,),
        in_specs=[
            pl.BlockSpec((rb, d), lambda i: (i, 0)),
            pl.BlockSpec((rb, d), lambda i: (i, 0)),
            pl.BlockSpec((rb, d), lambda i: (i, 0)),
            pl.BlockSpec((rb, CW), lambda i: (i, 0)),
            pl.BlockSpec((rb, CW), lambda i: (i, 0)),
            pl.BlockSpec((1, d), lambda i: (0, 0)),
        ],
        out_specs=pl.BlockSpec((1, d), lambda i: (0, 0)),
        out_shape=jax.ShapeDtypeStruct((1, d), jnp.float32),
    )(s0, s1, g2, c0, c1, b2)


def kernel(x, edge_index, rootIndex, W1, b1, W2, b2):
    n, d = x.shape
    src = edge_index[0]
    dst = edge_index[1]

    counts = _sc_count_dst(dst, n)
    c0, c1 = counts[0], counts[1]
    g1 = _tc_pre(x, W1, c0, c1)
    parts1 = _sc_scatter_rows(g1, src, dst)
    xroot = lax.dynamic_slice_in_dim(x, rootIndex, 1, axis=0)
    conv1, g2 = _tc_mid(parts1[0], parts1[1], g1, c0, c1,
                        b1.reshape(1, -1), xroot, W2)
    parts2 = _sc_scatter_rows(g2, src, dst)
    mean2 = _tc_final(parts2[0], parts2[1], g2, c0, c1, b2.reshape(1, -1))
    root1 = lax.dynamic_slice_in_dim(conv1, rootIndex, 1, axis=0)
    return jnp.concatenate([root1, mean2], axis=1)
